# baseline (speedup 1.0000x reference)
import functools

import jax
import jax.numpy as jnp
from jax import lax
from jax.experimental import pallas as pl
from jax.experimental.pallas import tpu as pltpu
from jax.experimental.pallas import tpu_sc as plsc

NC = 2
NS = 16
NW = NC * NS
LANES = 16

N = 10000
D = 128
E = 320000

EPW = E // NW
K = 80
NCHUNK = EPW // K
N2 = 10240
RPT = N2 // NS

BN = 2000
BE = 64000


def _mesh():
    return plsc.VectorSubcoreMesh(core_axis_name="c", subcore_axis_name="s")


@functools.partial(
    pl.kernel,
    out_type=jax.ShapeDtypeStruct((NW, 1, N), jnp.float32),
    mesh=_mesh(),
    scratch_types=[
        pltpu.VMEM((EPW,), jnp.int32),
        pltpu.VMEM((N,), jnp.float32),
    ],
    compiler_params=pltpu.CompilerParams(needs_layout_passes=False),
)
def _deg_kernel(dst_hbm, out_hbm, idx_v, acc_v):
    cid = lax.axis_index("c")
    sid = lax.axis_index("s")
    wid = sid * NC + cid

    zeros16 = jnp.zeros((LANES,), jnp.float32)

    def zero_body(i, carry):
        acc_v[pl.ds(i * LANES, LANES)] = zeros16
        return carry

    lax.fori_loop(0, N // LANES, zero_body, 0)

    pltpu.sync_copy(dst_hbm.at[pl.ds(wid * EPW, EPW)], idx_v)

    ones16 = jnp.ones((LANES,), jnp.float32)

    def count_body(i, carry):
        idx = idx_v[pl.ds(i * LANES, LANES)]
        plsc.addupdate_scatter(acc_v, [idx], ones16)
        return carry

    lax.fori_loop(0, EPW // LANES, count_body, 0)

    pltpu.sync_copy(acc_v, out_hbm.at[wid, 0])


NB = 4
GA = 3
NMAIN = (NCHUNK // NB) * NB
ZR = 40


@functools.partial(
    pl.kernel,
    out_type=jax.ShapeDtypeStruct((NC, N2, D), jnp.float32),
    mesh=_mesh(),
    scratch_types=[
        [pltpu.VMEM((K,), jnp.int32)] * NB,
        [pltpu.VMEM((K,), jnp.int32)] * NB,
        [pltpu.VMEM((K,), jnp.int32)] * NB,
        pltpu.VMEM((NB, K, D), jnp.float32),
        pltpu.VMEM((ZR, D), jnp.float32),
        pltpu.VMEM_SHARED((N2, D), jnp.float32),
        [pltpu.SemaphoreType.DMA] * NB,
        [pltpu.SemaphoreType.DMA] * NB,
        [pltpu.SemaphoreType.DMA] * NB,
        pltpu.SemaphoreType.DMA,
    ],
    compiler_params=pltpu.CompilerParams(needs_layout_passes=False),
)
def _agg_kernel(yhat_hbm, src_hbm, dst_hbm, out_hbm,
                sidx_v, didx_v, dscat_v, rows_v, zbuf_v, acc_sh,
                isems, gsems, ssems, zsem):
    cid = lax.axis_index("c")
    sid = lax.axis_index("s")
    wid = sid * NC + cid
    base = wid * EPW

    def idx_fetch(j, b):
        off = base + j * K
        return (pltpu.make_async_copy(
                    src_hbm.at[pl.ds(off, K)], sidx_v[b], isems[b]),
                pltpu.make_async_copy(
                    dst_hbm.at[pl.ds(off, K)], didx_v[b], isems[b]))

    class _Pair:
        def __init__(self, *ds):
            self.ds = ds

        def start(self):
            for d_ in self.ds:
                d_.start()

        def wait(self):
            for d_ in self.ds:
                d_.wait()

    def gather(b):
        return _Pair(
            pltpu.make_async_copy(
                yhat_hbm.at[sidx_v[b].at[pl.ds(0, K // 2)]],
                rows_v.at[b, pl.ds(0, K // 2)], gsems[b]),
            pltpu.make_async_copy(
                yhat_hbm.at[sidx_v[b].at[pl.ds(K // 2, K // 2)]],
                rows_v.at[b, pl.ds(K // 2, K // 2)], gsems[b]))

    def scat_desc(b):
        return pltpu.make_async_copy(
            rows_v.at[b], acc_sh.at[dscat_v[b]], ssems[b])

    stripe = pl.ds(sid * RPT, RPT)
    zeros16 = jnp.zeros((LANES,), jnp.float32)
    for r in range(ZR):
        for l in range(D // LANES):
            zbuf_v[r, pl.ds(l * LANES, LANES)] = zeros16
    zcopies = [
        pltpu.make_async_copy(
            zbuf_v, acc_sh.at[pl.ds(sid * RPT + t * ZR, ZR)], zsem)
        for t in range(RPT // ZR)
    ]
    for z in zcopies:
        z.start()

    for b in range(NB):
        for d_ in idx_fetch(b, b):
            d_.start()

    for g in range(GA):
        for d_ in idx_fetch(g, g):
            d_.wait()
        gather(g).start()

    for z in zcopies:
        z.wait()
    plsc.subcore_barrier()

    def outer(jj, carry):
        j0 = jj * NB
        for b in range(NB):
            j = j0 + b
            bg = (b + GA) % NB

            @pl.when(j + GA < NMAIN)
            def _():
                for d_ in idx_fetch(j + GA, bg):
                    d_.wait()

                @pl.when(j + GA >= NB)
                def _():
                    scat_desc(bg).wait()

                gather(bg).start()

            gather(b).wait()
            for l in range(K // LANES):
                dscat_v[b][pl.ds(l * LANES, LANES)] = (
                    didx_v[b][pl.ds(l * LANES, LANES)])
            pltpu.async_copy(rows_v.at[b], acc_sh.at[dscat_v[b]], ssems[b],
                             add=True)

            @pl.when(j + NB < NMAIN)
            def _():
                for d_ in idx_fetch(j + NB, b):
                    d_.start()
        return carry

    lax.fori_loop(0, NMAIN // NB, outer, 0)

    for b in range(NB):
        scat_desc(b).wait()

    for j in range(NMAIN, NCHUNK):
        for d_ in idx_fetch(j, 0):
            d_.start()
        for d_ in idx_fetch(j, 0):
            d_.wait()
        gather(0).start()
        gather(0).wait()
        pltpu.sync_copy(rows_v.at[0], acc_sh.at[didx_v[0]], add=True)

    plsc.subcore_barrier()
    pltpu.sync_copy(acc_sh.at[stripe], out_hbm.at[cid, stripe])


def _degred_body(degp_ref, dis_ref):
    deg = jnp.sum(degp_ref[...], axis=(0, 1)) + 1.0
    dis_ref[...] = lax.rsqrt(deg)[None, :]


def _degred(degp):
    return pl.pallas_call(
        _degred_body,
        out_shape=jax.ShapeDtypeStruct((1, N), jnp.float32),
    )(degp)


def _slice_body(ei_ref, src_ref, dst_ref):
    src_ref[...] = ei_ref[0]
    dst_ref[...] = ei_ref[1]


def _slice_edges(ei):
    return pl.pallas_call(
        _slice_body,
        out_shape=[jax.ShapeDtypeStruct((E,), jnp.int32),
                   jax.ShapeDtypeStruct((E,), jnp.int32)],
    )(ei)


def _matmul_body(x_ref, w_ref, y_ref):
    y_ref[...] = jnp.dot(x_ref[...], w_ref[...],
                         preferred_element_type=jnp.float32)


def _matmul(x, w):
    grid = (N // BN,)
    return pl.pallas_call(
        _matmul_body,
        grid=grid,
        in_specs=[pl.BlockSpec((BN, D), lambda i: (i, 0)),
                  pl.BlockSpec((D, D), lambda i: (0, 0))],
        out_specs=pl.BlockSpec((BN, D), lambda i: (i, 0)),
        out_shape=jax.ShapeDtypeStruct((N, D), jnp.float32),
    )(x, w)


def _scale_body(y_ref, dis_ref, yhat_ref):
    yhat_ref[...] = y_ref[...] * dis_ref[...]


def _scale(y, dis):
    grid = (N // BN,)
    return pl.pallas_call(
        _scale_body,
        grid=grid,
        in_specs=[pl.BlockSpec((BN, D), lambda i: (i, 0)),
                  pl.BlockSpec((BN, 1), lambda i: (i, 0))],
        out_specs=pl.BlockSpec((BN, D), lambda i: (i, 0)),
        out_shape=jax.ShapeDtypeStruct((N, D), jnp.float32),
    )(y, dis)


def _mid_body(p_ref, yhat_prev_ref, dis_ref, b_ref, w_ref, yhat_ref):
    dis = dis_ref[...]
    h = dis * (p_ref[0] + p_ref[1] + yhat_prev_ref[...]) + b_ref[...]
    y = jnp.dot(h, w_ref[...], preferred_element_type=jnp.float32)
    yhat_ref[...] = y * dis


def _mid(p, yhat_prev, dis, b, w):
    grid = (N // BN,)
    return pl.pallas_call(
        _mid_body,
        grid=grid,
        in_specs=[pl.BlockSpec((NC, BN, D), lambda i: (0, i, 0)),
                  pl.BlockSpec((BN, D), lambda i: (i, 0)),
                  pl.BlockSpec((BN, 1), lambda i: (i, 0)),
                  pl.BlockSpec((1, D), lambda i: (0, 0)),
                  pl.BlockSpec((D, D), lambda i: (0, 0))],
        out_specs=pl.BlockSpec((BN, D), lambda i: (i, 0)),
        out_shape=jax.ShapeDtypeStruct((N, D), jnp.float32),
    )(p, yhat_prev, dis, b, w)


def _final_body(p_ref, yhat_ref, dis_ref, b_ref, out_ref):
    h = dis_ref[...] * (p_ref[0] + p_ref[1] + yhat_ref[...]) + b_ref[...]
    out_ref[...] = jnp.maximum(h, 0.0)


def _final(p, yhat, dis, b):
    grid = (N // BN,)
    return pl.pallas_call(
        _final_body,
        grid=grid,
        in_specs=[pl.BlockSpec((NC, BN, D), lambda i: (0, i, 0)),
                  pl.BlockSpec((BN, D), lambda i: (i, 0)),
                  pl.BlockSpec((BN, 1), lambda i: (i, 0)),
                  pl.BlockSpec((1, D), lambda i: (0, 0))],
        out_specs=pl.BlockSpec((BN, D), lambda i: (i, 0)),
        out_shape=jax.ShapeDtypeStruct((N, D), jnp.float32),
    )(p, yhat, dis, b)


def kernel(x, edge_index, W1, b1, W2, b2, W3, b3):
    assert x.shape == (N, D) and edge_index.shape == (2, E)

    src, dst = _slice_edges(edge_index)

    y1 = _matmul(x, W1)
    degp = _deg_kernel(dst)
    dis = _degred(degp).reshape(N, 1)
    yhat1 = _scale(y1, dis)

    p1 = _agg_kernel(yhat1, src, dst)
    yhat2 = _mid(p1, yhat1, dis, b1.reshape(1, D), W2)
    p2 = _agg_kernel(yhat2, src, dst)
    yhat3 = _mid(p2, yhat2, dis, b2.reshape(1, D), W3)
    p3 = _agg_kernel(yhat3, src, dst)
    return _final(p3, yhat3, dis, b3.reshape(1, D))


# speedup vs baseline: 1.0016x; 1.0016x over previous
import functools

import jax
import jax.numpy as jnp
from jax import lax
from jax.experimental import pallas as pl
from jax.experimental.pallas import tpu as pltpu
from jax.experimental.pallas import tpu_sc as plsc

NC = 2
NS = 16
NW = NC * NS
LANES = 16

N = 10000
D = 128
E = 320000

EPW = E // NW
K = 80
NCHUNK = EPW // K
N2 = 10240
RPT = N2 // NS

BN = 2000
BE = 64000


def _mesh():
    return plsc.VectorSubcoreMesh(core_axis_name="c", subcore_axis_name="s")


@functools.partial(
    pl.kernel,
    out_type=jax.ShapeDtypeStruct((NW, 1, N), jnp.float32),
    mesh=_mesh(),
    scratch_types=[
        pltpu.VMEM((EPW,), jnp.int32),
        pltpu.VMEM((N,), jnp.float32),
    ],
    compiler_params=pltpu.CompilerParams(needs_layout_passes=False),
)
def _deg_kernel(dst_hbm, out_hbm, idx_v, acc_v):
    cid = lax.axis_index("c")
    sid = lax.axis_index("s")
    wid = sid * NC + cid

    zeros16 = jnp.zeros((LANES,), jnp.float32)

    def zero_body(i, carry):
        acc_v[pl.ds(i * LANES, LANES)] = zeros16
        return carry

    lax.fori_loop(0, N // LANES, zero_body, 0)

    pltpu.sync_copy(dst_hbm.at[pl.ds(wid * EPW, EPW)], idx_v)

    ones16 = jnp.ones((LANES,), jnp.float32)

    def count_body(i, carry):
        idx = idx_v[pl.ds(i * LANES, LANES)]
        plsc.addupdate_scatter(acc_v, [idx], ones16)
        return carry

    lax.fori_loop(0, EPW // LANES, count_body, 0)

    pltpu.sync_copy(acc_v, out_hbm.at[wid, 0])


NB = 4
GA = 3
NMAIN = (NCHUNK // NB) * NB
ZR = 40


@functools.partial(
    pl.kernel,
    out_type=jax.ShapeDtypeStruct((NC, N2, D), jnp.float32),
    mesh=_mesh(),
    scratch_types=[
        [pltpu.VMEM((K,), jnp.int32)] * NB,
        [pltpu.VMEM((K,), jnp.int32)] * NB,
        [pltpu.VMEM((K,), jnp.int32)] * NB,
        pltpu.VMEM((NB, K, D), jnp.float32),
        pltpu.VMEM((ZR, D), jnp.float32),
        pltpu.VMEM_SHARED((N2, D), jnp.float32),
        [pltpu.SemaphoreType.DMA] * NB,
        [pltpu.SemaphoreType.DMA] * NB,
        [pltpu.SemaphoreType.DMA] * NB,
        pltpu.SemaphoreType.DMA,
    ],
    compiler_params=pltpu.CompilerParams(needs_layout_passes=False),
)
def _agg_kernel(yhat_hbm, src_hbm, dst_hbm, out_hbm,
                sidx_v, didx_v, dscat_v, rows_v, zbuf_v, acc_sh,
                isems, gsems, ssems, zsem):
    cid = lax.axis_index("c")
    sid = lax.axis_index("s")
    wid = sid * NC + cid
    base = wid * EPW

    def idx_fetch(j, b):
        off = base + j * K
        return (pltpu.make_async_copy(
                    src_hbm.at[pl.ds(off, K)], sidx_v[b], isems[b]),
                pltpu.make_async_copy(
                    dst_hbm.at[pl.ds(off, K)], didx_v[b], isems[b]))

    def gather(b):
        return pltpu.make_async_copy(
            yhat_hbm.at[sidx_v[b]], rows_v.at[b], gsems[b])

    def scat_desc(b):
        return pltpu.make_async_copy(
            rows_v.at[b], acc_sh.at[dscat_v[b]], ssems[b])

    stripe = pl.ds(sid * RPT, RPT)
    zeros16 = jnp.zeros((LANES,), jnp.float32)
    for r in range(ZR):
        for l in range(D // LANES):
            zbuf_v[r, pl.ds(l * LANES, LANES)] = zeros16
    zcopies = [
        pltpu.make_async_copy(
            zbuf_v, acc_sh.at[pl.ds(sid * RPT + t * ZR, ZR)], zsem)
        for t in range(RPT // ZR)
    ]
    for z in zcopies:
        z.start()

    for b in range(NB):
        for d_ in idx_fetch(b, b):
            d_.start()

    for g in range(GA):
        for d_ in idx_fetch(g, g):
            d_.wait()
        gather(g).start()

    for z in zcopies:
        z.wait()
    plsc.subcore_barrier()

    def outer(jj, carry):
        j0 = jj * NB
        for b in range(NB):
            j = j0 + b
            bg = (b + GA) % NB

            @pl.when(j + GA < NMAIN)
            def _():
                for d_ in idx_fetch(j + GA, bg):
                    d_.wait()

                @pl.when(j + GA >= NB)
                def _():
                    scat_desc(bg).wait()

                gather(bg).start()

            gather(b).wait()
            for l in range(K // LANES):
                dscat_v[b][pl.ds(l * LANES, LANES)] = (
                    didx_v[b][pl.ds(l * LANES, LANES)])
            pltpu.async_copy(rows_v.at[b], acc_sh.at[dscat_v[b]], ssems[b],
                             add=True, priority=1)

            @pl.when(j + NB < NMAIN)
            def _():
                for d_ in idx_fetch(j + NB, b):
                    d_.start()
        return carry

    lax.fori_loop(0, NMAIN // NB, outer, 0)

    for b in range(NB):
        scat_desc(b).wait()

    for j in range(NMAIN, NCHUNK):
        for d_ in idx_fetch(j, 0):
            d_.start()
        for d_ in idx_fetch(j, 0):
            d_.wait()
        gather(0).start()
        gather(0).wait()
        pltpu.sync_copy(rows_v.at[0], acc_sh.at[didx_v[0]], add=True)

    plsc.subcore_barrier()
    pltpu.sync_copy(acc_sh.at[stripe], out_hbm.at[cid, stripe])


def _degred_body(degp_ref, dis_ref):
    deg = jnp.sum(degp_ref[...], axis=(0, 1)) + 1.0
    dis_ref[...] = lax.rsqrt(deg)[None, :]


def _degred(degp):
    return pl.pallas_call(
        _degred_body,
        out_shape=jax.ShapeDtypeStruct((1, N), jnp.float32),
    )(degp)


def _slice_body(ei_ref, src_ref, dst_ref):
    src_ref[...] = ei_ref[0]
    dst_ref[...] = ei_ref[1]


def _slice_edges(ei):
    return pl.pallas_call(
        _slice_body,
        out_shape=[jax.ShapeDtypeStruct((E,), jnp.int32),
                   jax.ShapeDtypeStruct((E,), jnp.int32)],
    )(ei)


def _matmul_body(x_ref, w_ref, y_ref):
    y_ref[...] = jnp.dot(x_ref[...], w_ref[...],
                         preferred_element_type=jnp.float32)


def _matmul(x, w):
    grid = (N // BN,)
    return pl.pallas_call(
        _matmul_body,
        grid=grid,
        in_specs=[pl.BlockSpec((BN, D), lambda i: (i, 0)),
                  pl.BlockSpec((D, D), lambda i: (0, 0))],
        out_specs=pl.BlockSpec((BN, D), lambda i: (i, 0)),
        out_shape=jax.ShapeDtypeStruct((N, D), jnp.float32),
    )(x, w)


def _scale_body(y_ref, dis_ref, yhat_ref):
    yhat_ref[...] = y_ref[...] * dis_ref[...]


def _scale(y, dis):
    grid = (N // BN,)
    return pl.pallas_call(
        _scale_body,
        grid=grid,
        in_specs=[pl.BlockSpec((BN, D), lambda i: (i, 0)),
                  pl.BlockSpec((BN, 1), lambda i: (i, 0))],
        out_specs=pl.BlockSpec((BN, D), lambda i: (i, 0)),
        out_shape=jax.ShapeDtypeStruct((N, D), jnp.float32),
    )(y, dis)


def _mid_body(p_ref, yhat_prev_ref, dis_ref, b_ref, w_ref, yhat_ref):
    dis = dis_ref[...]
    h = dis * (p_ref[0] + p_ref[1] + yhat_prev_ref[...]) + b_ref[...]
    y = jnp.dot(h, w_ref[...], preferred_element_type=jnp.float32)
    yhat_ref[...] = y * dis


def _mid(p, yhat_prev, dis, b, w):
    grid = (N // BN,)
    return pl.pallas_call(
        _mid_body,
        grid=grid,
        in_specs=[pl.BlockSpec((NC, BN, D), lambda i: (0, i, 0)),
                  pl.BlockSpec((BN, D), lambda i: (i, 0)),
                  pl.BlockSpec((BN, 1), lambda i: (i, 0)),
                  pl.BlockSpec((1, D), lambda i: (0, 0)),
                  pl.BlockSpec((D, D), lambda i: (0, 0))],
        out_specs=pl.BlockSpec((BN, D), lambda i: (i, 0)),
        out_shape=jax.ShapeDtypeStruct((N, D), jnp.float32),
    )(p, yhat_prev, dis, b, w)


def _final_body(p_ref, yhat_ref, dis_ref, b_ref, out_ref):
    h = dis_ref[...] * (p_ref[0] + p_ref[1] + yhat_ref[...]) + b_ref[...]
    out_ref[...] = jnp.maximum(h, 0.0)


def _final(p, yhat, dis, b):
    grid = (N // BN,)
    return pl.pallas_call(
        _final_body,
        grid=grid,
        in_specs=[pl.BlockSpec((NC, BN, D), lambda i: (0, i, 0)),
                  pl.BlockSpec((BN, D), lambda i: (i, 0)),
                  pl.BlockSpec((BN, 1), lambda i: (i, 0)),
                  pl.BlockSpec((1, D), lambda i: (0, 0))],
        out_specs=pl.BlockSpec((BN, D), lambda i: (i, 0)),
        out_shape=jax.ShapeDtypeStruct((N, D), jnp.float32),
    )(p, yhat, dis, b)


def kernel(x, edge_index, W1, b1, W2, b2, W3, b3):
    assert x.shape == (N, D) and edge_index.shape == (2, E)

    src, dst = _slice_edges(edge_index)

    y1 = _matmul(x, W1)
    degp = _deg_kernel(dst)
    dis = _degred(degp).reshape(N, 1)
    yhat1 = _scale(y1, dis)

    p1 = _agg_kernel(yhat1, src, dst)
    yhat2 = _mid(p1, yhat1, dis, b1.reshape(1, D), W2)
    p2 = _agg_kernel(yhat2, src, dst)
    yhat3 = _mid(p2, yhat2, dis, b2.reshape(1, D), W3)
    p3 = _agg_kernel(yhat3, src, dst)
    return _final(p3, yhat3, dis, b3.reshape(1, D))


# speedup vs baseline: 1.0097x; 1.0082x over previous
import functools

import jax
import jax.numpy as jnp
from jax import lax
from jax.experimental import pallas as pl
from jax.experimental.pallas import tpu as pltpu
from jax.experimental.pallas import tpu_sc as plsc

NC = 2
NS = 16
NW = NC * NS
LANES = 16

N = 10000
D = 128
E = 320000

EPW = E // NW
K = 80
NCHUNK = EPW // K
N2 = 10240
RPT = N2 // NS

BN = 2000
BE = 64000


def _mesh():
    return plsc.VectorSubcoreMesh(core_axis_name="c", subcore_axis_name="s")


@functools.partial(
    pl.kernel,
    out_type=jax.ShapeDtypeStruct((NW, 1, N), jnp.float32),
    mesh=_mesh(),
    scratch_types=[
        pltpu.VMEM((EPW,), jnp.int32),
        pltpu.VMEM((N,), jnp.float32),
    ],
    compiler_params=pltpu.CompilerParams(needs_layout_passes=False),
)
def _deg_kernel(dst_hbm, out_hbm, idx_v, acc_v):
    cid = lax.axis_index("c")
    sid = lax.axis_index("s")
    wid = sid * NC + cid

    zeros16 = jnp.zeros((LANES,), jnp.float32)
    ZU = 5

    def zero_body(i, carry):
        for u in range(ZU):
            acc_v[pl.ds((i * ZU + u) * LANES, LANES)] = zeros16
        return carry

    lax.fori_loop(0, N // (LANES * ZU), zero_body, 0)

    pltpu.sync_copy(dst_hbm.at[pl.ds(wid * EPW, EPW)], idx_v)

    ones16 = jnp.ones((LANES,), jnp.float32)

    def count_body(i, carry):
        for u in range(ZU):
            idx = idx_v[pl.ds((i * ZU + u) * LANES, LANES)]
            plsc.addupdate_scatter(acc_v, [idx], ones16)
        return carry

    lax.fori_loop(0, EPW // (LANES * ZU), count_body, 0)

    pltpu.sync_copy(acc_v, out_hbm.at[wid, 0])


NB = 4
GA = 3
NMAIN = (NCHUNK // NB) * NB
ZR = 40


@functools.partial(
    pl.kernel,
    out_type=jax.ShapeDtypeStruct((NC, N2, D), jnp.float32),
    mesh=_mesh(),
    scratch_types=[
        [pltpu.VMEM((K,), jnp.int32)] * NB,
        [pltpu.VMEM((K,), jnp.int32)] * NB,
        [pltpu.VMEM((K,), jnp.int32)] * NB,
        pltpu.VMEM((NB, K, D), jnp.float32),
        pltpu.VMEM((ZR, D), jnp.float32),
        pltpu.VMEM_SHARED((N2, D), jnp.float32),
        [pltpu.SemaphoreType.DMA] * NB,
        [pltpu.SemaphoreType.DMA] * NB,
        [pltpu.SemaphoreType.DMA] * NB,
        pltpu.SemaphoreType.DMA,
    ],
    compiler_params=pltpu.CompilerParams(needs_layout_passes=False),
)
def _agg_kernel(yhat_hbm, src_hbm, dst_hbm, out_hbm,
                sidx_v, didx_v, dscat_v, rows_v, zbuf_v, acc_sh,
                isems, gsems, ssems, zsem):
    cid = lax.axis_index("c")
    sid = lax.axis_index("s")
    wid = sid * NC + cid
    base = wid * EPW

    def idx_fetch(j, b):
        off = base + j * K
        return (pltpu.make_async_copy(
                    src_hbm.at[pl.ds(off, K)], sidx_v[b], isems[b]),
                pltpu.make_async_copy(
                    dst_hbm.at[pl.ds(off, K)], didx_v[b], isems[b]))

    def gather(b):
        return pltpu.make_async_copy(
            yhat_hbm.at[sidx_v[b]], rows_v.at[b], gsems[b])

    def scat_desc(b):
        return pltpu.make_async_copy(
            rows_v.at[b], acc_sh.at[dscat_v[b]], ssems[b])

    stripe = pl.ds(sid * RPT, RPT)
    zeros16 = jnp.zeros((LANES,), jnp.float32)
    for r in range(ZR):
        for l in range(D // LANES):
            zbuf_v[r, pl.ds(l * LANES, LANES)] = zeros16
    zcopies = [
        pltpu.make_async_copy(
            zbuf_v, acc_sh.at[pl.ds(sid * RPT + t * ZR, ZR)], zsem)
        for t in range(RPT // ZR)
    ]
    for z in zcopies:
        z.start()

    for b in range(NB):
        for d_ in idx_fetch(b, b):
            d_.start()

    for g in range(GA):
        for d_ in idx_fetch(g, g):
            d_.wait()
        gather(g).start()

    for z in zcopies:
        z.wait()
    plsc.subcore_barrier()

    def outer(jj, carry):
        j0 = jj * NB
        for b in range(NB):
            j = j0 + b
            bg = (b + GA) % NB

            @pl.when(j + GA < NMAIN)
            def _():
                for d_ in idx_fetch(j + GA, bg):
                    d_.wait()

                @pl.when(j + GA >= NB)
                def _():
                    scat_desc(bg).wait()

                gather(bg).start()

            gather(b).wait()
            for l in range(K // LANES):
                dscat_v[b][pl.ds(l * LANES, LANES)] = (
                    didx_v[b][pl.ds(l * LANES, LANES)])
            pltpu.async_copy(rows_v.at[b], acc_sh.at[dscat_v[b]], ssems[b],
                             add=True)

            @pl.when(j + NB < NMAIN)
            def _():
                for d_ in idx_fetch(j + NB, b):
                    d_.start()
        return carry

    lax.fori_loop(0, NMAIN // NB, outer, 0)

    for b in range(NB):
        scat_desc(b).wait()

    for j in range(NMAIN, NCHUNK):
        for d_ in idx_fetch(j, 0):
            d_.start()
        for d_ in idx_fetch(j, 0):
            d_.wait()
        gather(0).start()
        gather(0).wait()
        pltpu.sync_copy(rows_v.at[0], acc_sh.at[didx_v[0]], add=True)

    plsc.subcore_barrier()
    pltpu.sync_copy(acc_sh.at[stripe], out_hbm.at[cid, stripe])


def _degred_body(degp_ref, dis_ref):
    deg = jnp.sum(degp_ref[...], axis=(0, 1)) + 1.0
    dis_ref[...] = lax.rsqrt(deg)[None, :]


def _degred(degp):
    return pl.pallas_call(
        _degred_body,
        out_shape=jax.ShapeDtypeStruct((1, N), jnp.float32),
    )(degp)


def _slice_body(ei_ref, src_ref, dst_ref):
    src_ref[...] = ei_ref[0]
    dst_ref[...] = ei_ref[1]


def _slice_edges(ei):
    return pl.pallas_call(
        _slice_body,
        out_shape=[jax.ShapeDtypeStruct((E,), jnp.int32),
                   jax.ShapeDtypeStruct((E,), jnp.int32)],
    )(ei)


def _matmul_body(x_ref, w_ref, y_ref):
    y_ref[...] = jnp.dot(x_ref[...], w_ref[...],
                         preferred_element_type=jnp.float32)


def _matmul(x, w):
    grid = (N // BN,)
    return pl.pallas_call(
        _matmul_body,
        grid=grid,
        in_specs=[pl.BlockSpec((BN, D), lambda i: (i, 0)),
                  pl.BlockSpec((D, D), lambda i: (0, 0))],
        out_specs=pl.BlockSpec((BN, D), lambda i: (i, 0)),
        out_shape=jax.ShapeDtypeStruct((N, D), jnp.float32),
    )(x, w)


def _scale_body(y_ref, dis_ref, yhat_ref):
    yhat_ref[...] = y_ref[...] * dis_ref[...]


def _scale(y, dis):
    grid = (N // BN,)
    return pl.pallas_call(
        _scale_body,
        grid=grid,
        in_specs=[pl.BlockSpec((BN, D), lambda i: (i, 0)),
                  pl.BlockSpec((BN, 1), lambda i: (i, 0))],
        out_specs=pl.BlockSpec((BN, D), lambda i: (i, 0)),
        out_shape=jax.ShapeDtypeStruct((N, D), jnp.float32),
    )(y, dis)


def _mid_body(p_ref, yhat_prev_ref, dis_ref, b_ref, w_ref, yhat_ref):
    dis = dis_ref[...]
    h = dis * (p_ref[0] + p_ref[1] + yhat_prev_ref[...]) + b_ref[...]
    y = jnp.dot(h, w_ref[...], preferred_element_type=jnp.float32)
    yhat_ref[...] = y * dis


def _mid(p, yhat_prev, dis, b, w):
    grid = (N // BN,)
    return pl.pallas_call(
        _mid_body,
        grid=grid,
        in_specs=[pl.BlockSpec((NC, BN, D), lambda i: (0, i, 0)),
                  pl.BlockSpec((BN, D), lambda i: (i, 0)),
                  pl.BlockSpec((BN, 1), lambda i: (i, 0)),
                  pl.BlockSpec((1, D), lambda i: (0, 0)),
                  pl.BlockSpec((D, D), lambda i: (0, 0))],
        out_specs=pl.BlockSpec((BN, D), lambda i: (i, 0)),
        out_shape=jax.ShapeDtypeStruct((N, D), jnp.float32),
    )(p, yhat_prev, dis, b, w)


def _final_body(p_ref, yhat_ref, dis_ref, b_ref, out_ref):
    h = dis_ref[...] * (p_ref[0] + p_ref[1] + yhat_ref[...]) + b_ref[...]
    out_ref[...] = jnp.maximum(h, 0.0)


def _final(p, yhat, dis, b):
    grid = (N // BN,)
    return pl.pallas_call(
        _final_body,
        grid=grid,
        in_specs=[pl.BlockSpec((NC, BN, D), lambda i: (0, i, 0)),
                  pl.BlockSpec((BN, D), lambda i: (i, 0)),
                  pl.BlockSpec((BN, 1), lambda i: (i, 0)),
                  pl.BlockSpec((1, D), lambda i: (0, 0))],
        out_specs=pl.BlockSpec((BN, D), lambda i: (i, 0)),
        out_shape=jax.ShapeDtypeStruct((N, D), jnp.float32),
    )(p, yhat, dis, b)


def kernel(x, edge_index, W1, b1, W2, b2, W3, b3):
    assert x.shape == (N, D) and edge_index.shape == (2, E)

    src, dst = _slice_edges(edge_index)

    y1 = _matmul(x, W1)
    degp = _deg_kernel(dst)
    dis = _degred(degp).reshape(N, 1)
    yhat1 = _scale(y1, dis)

    p1 = _agg_kernel(yhat1, src, dst)
    yhat2 = _mid(p1, yhat1, dis, b1.reshape(1, D), W2)
    p2 = _agg_kernel(yhat2, src, dst)
    yhat3 = _mid(p2, yhat2, dis, b2.reshape(1, D), W3)
    p3 = _agg_kernel(yhat3, src, dst)
    return _final(p3, yhat3, dis, b3.reshape(1, D))


# speedup vs baseline: 1.0202x; 1.0104x over previous
import functools

import jax
import jax.numpy as jnp
from jax import lax
from jax.experimental import pallas as pl
from jax.experimental.pallas import tpu as pltpu
from jax.experimental.pallas import tpu_sc as plsc

NC = 2
NS = 16
NW = NC * NS
LANES = 16

N = 10000
D = 128
E = 320000

EPW = E // NW
K = 80
NCHUNK = EPW // K
N2 = 10240
RPT = N2 // NS

BN = 5000
BE = 64000


def _mesh():
    return plsc.VectorSubcoreMesh(core_axis_name="c", subcore_axis_name="s")


@functools.partial(
    pl.kernel,
    out_type=jax.ShapeDtypeStruct((NW, 1, N), jnp.float32),
    mesh=_mesh(),
    scratch_types=[
        pltpu.VMEM((EPW,), jnp.int32),
        pltpu.VMEM((N,), jnp.float32),
    ],
    compiler_params=pltpu.CompilerParams(needs_layout_passes=False),
)
def _deg_kernel(dst_hbm, out_hbm, idx_v, acc_v):
    cid = lax.axis_index("c")
    sid = lax.axis_index("s")
    wid = sid * NC + cid

    zeros16 = jnp.zeros((LANES,), jnp.float32)
    ZU = 5

    def zero_body(i, carry):
        for u in range(ZU):
            acc_v[pl.ds((i * ZU + u) * LANES, LANES)] = zeros16
        return carry

    lax.fori_loop(0, N // (LANES * ZU), zero_body, 0)

    pltpu.sync_copy(dst_hbm.at[pl.ds(wid * EPW, EPW)], idx_v)

    ones16 = jnp.ones((LANES,), jnp.float32)

    def count_body(i, carry):
        for u in range(ZU):
            idx = idx_v[pl.ds((i * ZU + u) * LANES, LANES)]
            plsc.addupdate_scatter(acc_v, [idx], ones16)
        return carry

    lax.fori_loop(0, EPW // (LANES * ZU), count_body, 0)

    pltpu.sync_copy(acc_v, out_hbm.at[wid, 0])


NB = 4
GA = 3
NMAIN = (NCHUNK // NB) * NB
ZR = 40


@functools.partial(
    pl.kernel,
    out_type=jax.ShapeDtypeStruct((NC, N2, D), jnp.float32),
    mesh=_mesh(),
    scratch_types=[
        [pltpu.VMEM((K,), jnp.int32)] * NB,
        [pltpu.VMEM((K,), jnp.int32)] * NB,
        [pltpu.VMEM((K,), jnp.int32)] * NB,
        pltpu.VMEM((NB, K, D), jnp.float32),
        pltpu.VMEM((ZR, D), jnp.float32),
        pltpu.VMEM_SHARED((N2, D), jnp.float32),
        [pltpu.SemaphoreType.DMA] * NB,
        [pltpu.SemaphoreType.DMA] * NB,
        [pltpu.SemaphoreType.DMA] * NB,
        pltpu.SemaphoreType.DMA,
    ],
    compiler_params=pltpu.CompilerParams(needs_layout_passes=False),
)
def _agg_kernel(yhat_hbm, src_hbm, dst_hbm, out_hbm,
                sidx_v, didx_v, dscat_v, rows_v, zbuf_v, acc_sh,
                isems, gsems, ssems, zsem):
    cid = lax.axis_index("c")
    sid = lax.axis_index("s")
    wid = sid * NC + cid
    base = wid * EPW

    def idx_fetch(j, b):
        off = base + j * K
        return (pltpu.make_async_copy(
                    src_hbm.at[pl.ds(off, K)], sidx_v[b], isems[b]),
                pltpu.make_async_copy(
                    dst_hbm.at[pl.ds(off, K)], didx_v[b], isems[b]))

    def gather(b):
        return pltpu.make_async_copy(
            yhat_hbm.at[sidx_v[b]], rows_v.at[b], gsems[b])

    def scat_desc(b):
        return pltpu.make_async_copy(
            rows_v.at[b], acc_sh.at[dscat_v[b]], ssems[b])

    stripe = pl.ds(sid * RPT, RPT)
    zeros16 = jnp.zeros((LANES,), jnp.float32)
    for r in range(ZR):
        for l in range(D // LANES):
            zbuf_v[r, pl.ds(l * LANES, LANES)] = zeros16
    zcopies = [
        pltpu.make_async_copy(
            zbuf_v, acc_sh.at[pl.ds(sid * RPT + t * ZR, ZR)], zsem)
        for t in range(RPT // ZR)
    ]
    for z in zcopies:
        z.start()

    for b in range(NB):
        for d_ in idx_fetch(b, b):
            d_.start()

    for g in range(GA):
        for d_ in idx_fetch(g, g):
            d_.wait()
        gather(g).start()

    for z in zcopies:
        z.wait()
    plsc.subcore_barrier()

    def outer(jj, carry):
        j0 = jj * NB
        for b in range(NB):
            j = j0 + b
            bg = (b + GA) % NB

            @pl.when(j + GA < NMAIN)
            def _():
                for d_ in idx_fetch(j + GA, bg):
                    d_.wait()

                @pl.when(j + GA >= NB)
                def _():
                    scat_desc(bg).wait()

                gather(bg).start()

            gather(b).wait()
            for l in range(K // LANES):
                dscat_v[b][pl.ds(l * LANES, LANES)] = (
                    didx_v[b][pl.ds(l * LANES, LANES)])
            pltpu.async_copy(rows_v.at[b], acc_sh.at[dscat_v[b]], ssems[b],
                             add=True)

            @pl.when(j + NB < NMAIN)
            def _():
                for d_ in idx_fetch(j + NB, b):
                    d_.start()
        return carry

    lax.fori_loop(0, NMAIN // NB, outer, 0)

    for b in range(NB):
        scat_desc(b).wait()

    for j in range(NMAIN, NCHUNK):
        for d_ in idx_fetch(j, 0):
            d_.start()
        for d_ in idx_fetch(j, 0):
            d_.wait()
        gather(0).start()
        gather(0).wait()
        pltpu.sync_copy(rows_v.at[0], acc_sh.at[didx_v[0]], add=True)

    plsc.subcore_barrier()
    pltpu.sync_copy(acc_sh.at[stripe], out_hbm.at[cid, stripe])


def _degred_body(degp_ref, dis_ref):
    deg = jnp.sum(degp_ref[...], axis=(0, 1)) + 1.0
    dis_ref[...] = lax.rsqrt(deg)[None, :]


def _degred(degp):
    return pl.pallas_call(
        _degred_body,
        out_shape=jax.ShapeDtypeStruct((1, N), jnp.float32),
    )(degp)


def _slice_body(ei_ref, src_ref, dst_ref):
    src_ref[...] = ei_ref[0]
    dst_ref[...] = ei_ref[1]


def _slice_edges(ei):
    return pl.pallas_call(
        _slice_body,
        out_shape=[jax.ShapeDtypeStruct((E,), jnp.int32),
                   jax.ShapeDtypeStruct((E,), jnp.int32)],
    )(ei)


def _matmul_body(x_ref, w_ref, y_ref):
    y_ref[...] = jnp.dot(x_ref[...], w_ref[...],
                         preferred_element_type=jnp.float32)


def _matmul(x, w):
    grid = (N // BN,)
    return pl.pallas_call(
        _matmul_body,
        grid=grid,
        in_specs=[pl.BlockSpec((BN, D), lambda i: (i, 0)),
                  pl.BlockSpec((D, D), lambda i: (0, 0))],
        out_specs=pl.BlockSpec((BN, D), lambda i: (i, 0)),
        out_shape=jax.ShapeDtypeStruct((N, D), jnp.float32),
    )(x, w)


def _scale_body(y_ref, dis_ref, yhat_ref):
    yhat_ref[...] = y_ref[...] * dis_ref[...]


def _scale(y, dis):
    grid = (N // BN,)
    return pl.pallas_call(
        _scale_body,
        grid=grid,
        in_specs=[pl.BlockSpec((BN, D), lambda i: (i, 0)),
                  pl.BlockSpec((BN, 1), lambda i: (i, 0))],
        out_specs=pl.BlockSpec((BN, D), lambda i: (i, 0)),
        out_shape=jax.ShapeDtypeStruct((N, D), jnp.float32),
    )(y, dis)


def _mid_body(p_ref, yhat_prev_ref, dis_ref, b_ref, w_ref, yhat_ref):
    dis = dis_ref[...]
    h = dis * (p_ref[0] + p_ref[1] + yhat_prev_ref[...]) + b_ref[...]
    y = jnp.dot(h, w_ref[...], preferred_element_type=jnp.float32)
    yhat_ref[...] = y * dis


def _mid(p, yhat_prev, dis, b, w):
    grid = (N // BN,)
    return pl.pallas_call(
        _mid_body,
        grid=grid,
        in_specs=[pl.BlockSpec((NC, BN, D), lambda i: (0, i, 0)),
                  pl.BlockSpec((BN, D), lambda i: (i, 0)),
                  pl.BlockSpec((BN, 1), lambda i: (i, 0)),
                  pl.BlockSpec((1, D), lambda i: (0, 0)),
                  pl.BlockSpec((D, D), lambda i: (0, 0))],
        out_specs=pl.BlockSpec((BN, D), lambda i: (i, 0)),
        out_shape=jax.ShapeDtypeStruct((N, D), jnp.float32),
    )(p, yhat_prev, dis, b, w)


def _final_body(p_ref, yhat_ref, dis_ref, b_ref, out_ref):
    h = dis_ref[...] * (p_ref[0] + p_ref[1] + yhat_ref[...]) + b_ref[...]
    out_ref[...] = jnp.maximum(h, 0.0)


def _final(p, yhat, dis, b):
    grid = (N // BN,)
    return pl.pallas_call(
        _final_body,
        grid=grid,
        in_specs=[pl.BlockSpec((NC, BN, D), lambda i: (0, i, 0)),
                  pl.BlockSpec((BN, D), lambda i: (i, 0)),
                  pl.BlockSpec((BN, 1), lambda i: (i, 0)),
                  pl.BlockSpec((1, D), lambda i: (0, 0))],
        out_specs=pl.BlockSpec((BN, D), lambda i: (i, 0)),
        out_shape=jax.ShapeDtypeStruct((N, D), jnp.float32),
    )(p, yhat, dis, b)


def kernel(x, edge_index, W1, b1, W2, b2, W3, b3):
    assert x.shape == (N, D) and edge_index.shape == (2, E)

    src, dst = _slice_edges(edge_index)

    y1 = _matmul(x, W1)
    degp = _deg_kernel(dst)
    dis = _degred(degp).reshape(N, 1)
    yhat1 = _scale(y1, dis)

    p1 = _agg_kernel(yhat1, src, dst)
    yhat2 = _mid(p1, yhat1, dis, b1.reshape(1, D), W2)
    p2 = _agg_kernel(yhat2, src, dst)
    yhat3 = _mid(p2, yhat2, dis, b2.reshape(1, D), W3)
    p3 = _agg_kernel(yhat3, src, dst)
    return _final(p3, yhat3, dis, b3.reshape(1, D))
